# baseline (speedup 1.0000x reference)
import functools

import jax
import jax.numpy as jnp
from jax import lax
from jax.experimental import pallas as pl
from jax.experimental.pallas import tpu as pltpu
from jax.experimental.pallas import tpu_sc as plsc

_NC = 2
_NS = 16
_CHUNK = 128


def _emb_lookup(table, labels2d, B, D):
    NW = _NC * _NS
    b_per_w = B // NW
    n_chunks = b_per_w // _CHUNK
    mesh = plsc.VectorSubcoreMesh(core_axis_name="c", subcore_axis_name="s")

    @functools.partial(
        pl.kernel,
        out_type=jax.ShapeDtypeStruct((B, D), jnp.float32),
        mesh=mesh,
        compiler_params=pltpu.CompilerParams(use_tc_tiling_on_sc=False),
        scratch_types=[
            pltpu.VMEM((n_chunks, _CHUNK), jnp.int32),
            pltpu.VMEM((b_per_w, D), jnp.float32),
            pltpu.SemaphoreType.DMA,
        ],
    )
    def body(table_hbm, idx_hbm, out_hbm, idx_v, rows_v, sem):
        wid = lax.axis_index("s") * _NC + lax.axis_index("c")
        base = wid * b_per_w
        row0 = wid * n_chunks
        pltpu.sync_copy(idx_hbm.at[pl.ds(row0, n_chunks)], idx_v)
        copies = []
        for j in range(n_chunks):
            copies.append(
                pltpu.async_copy(
                    table_hbm.at[idx_v.at[j]],
                    rows_v.at[pl.ds(j * _CHUNK, _CHUNK)],
                    sem,
                )
            )
        for c in copies:
            c.wait()
        pltpu.sync_copy(rows_v, out_hbm.at[pl.ds(base, b_per_w)])

    return body(table, labels2d)


def kernel(labels, embedding_table):
    B, = labels.shape
    V, D = embedding_table.shape
    labels2d = labels.astype(jnp.int32).reshape(B // _CHUNK, _CHUNK)
    return _emb_lookup(embedding_table, labels2d, B, D)


# speedup vs baseline: 1.4470x; 1.4470x over previous
import functools

import jax
import jax.numpy as jnp
from jax import lax
from jax.experimental import pallas as pl
from jax.experimental.pallas import tpu as pltpu
from jax.experimental.pallas import tpu_sc as plsc

_NC = 2
_NS = 16
_NW = _NC * _NS
_SEG = 31744
_CH = 256
_TAIL0 = _NW - 1
_LCAP = 16400


def _iota16():
    return lax.broadcasted_iota(jnp.int32, (16,), 0)


def _emb_lookup(tableC, labels, tail2, V, B):
    tail_r0 = _TAIL0 * _SEG + 62 * _CH
    n_tail = V - tail_r0
    mesh = plsc.VectorSubcoreMesh(core_axis_name="c", subcore_axis_name="s")

    @functools.partial(
        pl.kernel,
        out_type=jax.ShapeDtypeStruct((B + 32, 128), jnp.float32),
        mesh=mesh,
        compiler_params=pltpu.CompilerParams(use_tc_tiling_on_sc=True, needs_layout_passes=False),
        scratch_types=[
            pltpu.VMEM((2048,), jnp.int32),
            pltpu.VMEM((_LCAP,), jnp.int32),
            pltpu.VMEM((_LCAP,), jnp.int32),
            pltpu.VMEM((_LCAP,), jnp.int32),
            pltpu.VMEM((_LCAP,), jnp.int32),
            pltpu.VMEM((2, 8, 8, _CH), jnp.float32),
            pltpu.VMEM((128, 64), jnp.float32),
            pltpu.VMEM((64, 128), jnp.float32),
            pltpu.VMEM((64,), jnp.int32),
            pltpu.SemaphoreType.DMA,
            pltpu.SemaphoreType.DMA,
        ],
    )
    def body(tab, idx, tl, out, lab_blk, mlab, mb, clab, cb, ring, tailv,
             stag, bstage, sem_c, sem_s):
        wid = lax.axis_index("s") * _NC + lax.axis_index("c")
        lo = wid * _SEG
        n_ch = jnp.where(wid == _TAIL0, 62, 124)
        hi = lo + n_ch * _CH
        iota = _iota16()
        is0 = wid == 0

        def scan_block(blk, off):
            pltpu.sync_copy(idx.at[pl.ds(blk * 2048, 2048)], lab_blk)

            def scan_i(i, off):
                lab = lab_blk[pl.ds(i * 16, 16)]
                m = (lab >= lo) & (lab < hi)
                m = m | (is0 & (lab >= tail_r0))
                cnt = jnp.sum(jnp.where(m, 1, 0).astype(jnp.int32))
                plsc.store_compressed(mlab.at[pl.ds(off, 16)], lab, mask=m)
                bv = iota + (blk * 2048 + i * 16)
                plsc.store_compressed(mb.at[pl.ds(off, 16)], bv, mask=m)
                return off + cnt

            return lax.fori_loop(0, 128, scan_i, off)

        n_m = lax.fori_loop(0, 8, scan_block, 0)
        n_mg = (n_m + 15) // 16

        for k in range(4):
            bstage[pl.ds(k * 16, 16)] = iota + B

        def fire():
            pltpu.async_copy(stag, out.at[bstage], sem_s).wait()

        def process(gather_fn, r0, width, staged):
            def comp_j(j, cc):
                lab = mlab[pl.ds(j * 16, 16)]
                b = mb[pl.ds(j * 16, 16)]
                valid = (iota + j * 16) < n_m
                m2 = (lab >= r0) & (lab < r0 + width) & valid
                cnt = jnp.sum(jnp.where(m2, 1, 0).astype(jnp.int32))
                plsc.store_compressed(clab.at[pl.ds(cc, 16)], lab, mask=m2)
                plsc.store_compressed(cb.at[pl.ds(cc, 16)], b, mask=m2)
                return cc + cnt

            n_c = lax.fori_loop(0, n_mg, comp_j, 0)

            def grp(g, st):
                labv = clab[pl.ds(g * 16, 16)]
                bv = cb[pl.ds(g * 16, 16)]
                valid = (iota + g * 16) < n_c
                bvs = jnp.where(valid, bv, iota + B)
                rloc = jnp.minimum(jnp.maximum(labv - r0, 0), width - 1)
                row0 = st % 64
                bstage[pl.ds(row0, 16)] = bvs
                for i in range(16):
                    rs = jnp.broadcast_to(rloc[i], (16,))
                    rowv = jnp.broadcast_to(row0 + i, (16,))
                    for j in range(4):
                        d = iota + j * 16
                        vals = gather_fn(rs, d)
                        plsc.store_scatter(stag, [rowv, d], vals)
                st = st + 16

                @pl.when(st % 64 == 0)
                def _():
                    fire()

                return st

            return lax.fori_loop(0, (n_c + 15) // 16, grp, staged)

        pltpu.async_copy(
            tab.at[:, :, pl.ds(lo, _CH)], ring.at[0], sem_c
        )

        def chunk_body(c, staged):
            slot = lax.rem(c, 2)
            r0 = lo + c * _CH

            @pl.when(c + 1 < n_ch)
            def _():
                pltpu.async_copy(
                    tab.at[:, :, pl.ds(r0 + _CH, _CH)],
                    ring.at[lax.rem(c + 1, 2)],
                    sem_c,
                )

            pltpu.make_async_copy(
                tab.at[:, :, pl.ds(r0, _CH)], ring.at[slot], sem_c
            ).wait()

            def gather_ring(rs, d):
                return plsc.load_gather(ring.at[slot], [d >> 3, d & 7, rs])

            return process(gather_ring, r0, _CH, staged)

        staged = lax.fori_loop(0, n_ch, chunk_body, 0)

        @pl.when(is0)
        def _():
            pltpu.sync_copy(tl, tailv)

            def gather_tail(rs, d):
                return plsc.load_gather(tailv, [rs, d])

            process(gather_tail, tail_r0, n_tail, staged)

        fire()

    return body(tableC, labels, tail2)


def kernel(labels, embedding_table):
    B, = labels.shape
    V, D = embedding_table.shape
    tableC = embedding_table.T.reshape(8, D // 8, V)
    tail_r0 = _TAIL0 * _SEG + 62 * _CH
    tail2 = jnp.pad(embedding_table[tail_r0:], ((0, 128 - (V - tail_r0)), (0, 0)))
    out128 = _emb_lookup(tableC, labels.astype(jnp.int32), tail2, V, B)
    return out128[:B, :D]
